# baseline (speedup 1.0000x reference)
import functools

import jax
import jax.numpy as jnp
from jax import lax
from jax.experimental import pallas as pl
from jax.experimental.pallas import tpu as pltpu
from jax.experimental.pallas import tpu_sc as plsc

STEPS = 3

NC = 2
NS = 16
NW = NC * NS
CH = 128
MPAD = 8


def _row_block(n):
    for r in (1024, 1000, 512, 500, 256, 250, 200, 128, 125, 80, 40, 8, 1):
        if n % r == 0:
            return r
    return 1


def _init_body(nb, h_ref, w0_ref, b0_ref, whh_ref, bhh_ref, m_ref, gh_ref):
    i = pl.program_id(0)
    h = h_ref[...]

    @pl.when(i < nb)
    def _():
        m_ref[...] = (
            jnp.dot(h, w0_ref[...], preferred_element_type=jnp.float32)
            + b0_ref[...]
        )

    @pl.when(i >= nb)
    def _():
        m_ref[...] = jnp.zeros_like(m_ref)

    gh_ref[...] = (
        jnp.dot(h, whh_ref[...], preferred_element_type=jnp.float32)
        + bhh_ref[...]
    )


def _init(h, w0_t, b0_row, whh_t, bhh_row):
    n, d = h.shape
    r = _row_block(n)
    nb = n // r
    return pl.pallas_call(
        functools.partial(_init_body, nb),
        grid=(nb + 1,),
        in_specs=[
            pl.BlockSpec((r, d), lambda i: (jnp.minimum(i, nb - 1), 0)),
            pl.BlockSpec((d, d), lambda i: (0, 0)),
            pl.BlockSpec((1, d), lambda i: (0, 0)),
            pl.BlockSpec((d, 3 * d), lambda i: (0, 0)),
            pl.BlockSpec((1, 3 * d), lambda i: (0, 0)),
        ],
        out_specs=[
            pl.BlockSpec((r, d), lambda i: (i, 0)),
            pl.BlockSpec((r, 3 * d), lambda i: (jnp.minimum(i, nb - 1), 0)),
        ],
        out_shape=[
            jax.ShapeDtypeStruct((n + MPAD, d), jnp.float32),
            jax.ShapeDtypeStruct((n, 3 * d), jnp.float32),
        ],
    )(h, w0_t, b0_row, whh_t, bhh_row)


def _gates(a_ref, h_ref, gh_ref, wih_ref, bih_ref):
    a = a_ref[0] + a_ref[1]
    h = h_ref[...]
    d = h.shape[1]
    gi = jnp.dot(a, wih_ref[...], preferred_element_type=jnp.float32) + bih_ref[...]
    gh = gh_ref[...]
    r = jax.nn.sigmoid(gi[:, :d] + gh[:, :d])
    z = jax.nn.sigmoid(gi[:, d:2 * d] + gh[:, d:2 * d])
    nn = jnp.tanh(gi[:, 2 * d:] + r * gh[:, 2 * d:])
    return (1.0 - z) * nn + z * h


def _boundary_body(nb, a_ref, h_ref, gh_ref, wih_ref, bih_ref, w0_ref,
                   b0_ref, hn_ref, mn_ref):
    i = pl.program_id(0)
    hn = _gates(a_ref, h_ref, gh_ref, wih_ref, bih_ref)
    hn_ref[...] = hn

    @pl.when(i < nb)
    def _():
        mn_ref[...] = (
            jnp.dot(hn, w0_ref[...], preferred_element_type=jnp.float32)
            + b0_ref[...]
        )

    @pl.when(i >= nb)
    def _():
        mn_ref[...] = jnp.zeros_like(mn_ref)


def _boundary(a_parts, h, gh, wih_t, bih_row, w0_t, b0_row):
    n, d = h.shape
    r = _row_block(n)
    nb = n // r
    clamp = lambda i: (jnp.minimum(i, nb - 1), 0)
    return pl.pallas_call(
        functools.partial(_boundary_body, nb),
        grid=(nb + 1,),
        in_specs=[
            pl.BlockSpec((2, r, d), lambda i: (0, jnp.minimum(i, nb - 1), 0)),
            pl.BlockSpec((r, d), clamp),
            pl.BlockSpec((r, 3 * d), clamp),
            pl.BlockSpec((d, 3 * d), lambda i: (0, 0)),
            pl.BlockSpec((1, 3 * d), lambda i: (0, 0)),
            pl.BlockSpec((d, d), lambda i: (0, 0)),
            pl.BlockSpec((1, d), lambda i: (0, 0)),
        ],
        out_specs=[
            pl.BlockSpec((r, d), clamp),
            pl.BlockSpec((r, d), lambda i: (i, 0)),
        ],
        out_shape=[
            jax.ShapeDtypeStruct((n, d), jnp.float32),
            jax.ShapeDtypeStruct((n + MPAD, d), jnp.float32),
        ],
    )(a_parts, h, gh, wih_t, bih_row, w0_t, b0_row)


def _gh_body(h_ref, whh_ref, bhh_ref, o_ref):
    o_ref[...] = (
        jnp.dot(h_ref[...], whh_ref[...], preferred_element_type=jnp.float32)
        + bhh_ref[...]
    )


def _gh(h, whh_t, bhh_row):
    n, d = h.shape
    r = _row_block(n)
    return pl.pallas_call(
        _gh_body,
        grid=(n // r,),
        in_specs=[
            pl.BlockSpec((r, d), lambda i: (i, 0)),
            pl.BlockSpec((d, 3 * d), lambda i: (0, 0)),
            pl.BlockSpec((1, 3 * d), lambda i: (0, 0)),
        ],
        out_specs=pl.BlockSpec((r, 3 * d), lambda i: (i, 0)),
        out_shape=jax.ShapeDtypeStruct((n, 3 * d), jnp.float32),
    )(h, whh_t, bhh_row)


def _final_body(a_ref, h_ref, gh_ref, wih_ref, bih_ref, o_ref):
    o_ref[...] = _gates(a_ref, h_ref, gh_ref, wih_ref, bih_ref)


def _final_gru(a_parts, h, gh, wih_t, bih_row):
    n, d = h.shape
    r = _row_block(n)
    return pl.pallas_call(
        _final_body,
        grid=(n // r,),
        in_specs=[
            pl.BlockSpec((2, r, d), lambda i: (0, i, 0)),
            pl.BlockSpec((r, d), lambda i: (i, 0)),
            pl.BlockSpec((r, 3 * d), lambda i: (i, 0)),
            pl.BlockSpec((d, 3 * d), lambda i: (0, 0)),
            pl.BlockSpec((1, 3 * d), lambda i: (0, 0)),
        ],
        out_specs=pl.BlockSpec((r, d), lambda i: (i, 0)),
        out_shape=jax.ShapeDtypeStruct((n, d), jnp.float32),
    )(a_parts, h, gh, wih_t, bih_row)


@functools.lru_cache(maxsize=None)
def _make_sc_scatter(n, d, nchunks):
    cpw = nchunks // NW
    zr = 8
    for c in (80, 64, 40, 32, 16, 8):
        if n % c == 0:
            zr = c
            break
    nz = n // zr
    per_sub = -(-nz // NS)

    mesh = plsc.VectorSubcoreMesh(core_axis_name="c", subcore_axis_name="s")
    scratch = [
        pltpu.VMEM((2, CH), jnp.int32),
        pltpu.VMEM((2, CH), jnp.int32),
        pltpu.VMEM((CH, d), jnp.float32),
        pltpu.VMEM((CH, d), jnp.float32),
        pltpu.VMEM_SHARED((n, d), jnp.float32),
        pltpu.VMEM((zr, d), jnp.float32),
        pltpu.SemaphoreType.DMA,
        pltpu.SemaphoreType.DMA,
    ]

    def body(m_hbm, idx_hbm, out_hbm, ibuf_a, ibuf_b, rows_a, rows_b,
             acc, zbuf, sem_a, sem_b):
        cid = lax.axis_index("c")
        sid = lax.axis_index("s")
        wid = cid * NS + sid

        zero16 = jnp.zeros((16,), jnp.float32)

        def zrow(i, _):
            for c in range(d // 16):
                zbuf[i, pl.ds(c * 16, 16)] = zero16
            return 0

        lax.fori_loop(0, zr, zrow, 0)

        zdescs = []
        for t in range(per_sub):
            idx = lax.rem(sid + t * NS, nz)
            zdescs.append(
                pltpu.async_copy(zbuf, acc.at[pl.ds(idx * zr, zr)], sem_a))
        for dsc in zdescs:
            dsc.wait()
        plsc.subcore_barrier()

        cbase = wid * cpw

        def pair(i, _):
            t = cbase + 2 * i
            ia = pltpu.async_copy(idx_hbm.at[t], ibuf_a, sem_a)
            ib = pltpu.async_copy(idx_hbm.at[t + 1], ibuf_b, sem_b)
            ia.wait()
            ga = pltpu.async_copy(m_hbm.at[ibuf_a.at[0]], rows_a, sem_a)
            ib.wait()
            gb = pltpu.async_copy(m_hbm.at[ibuf_b.at[0]], rows_b, sem_b)
            ga.wait()
            pltpu.sync_copy(rows_a, acc.at[ibuf_a.at[1]], add=True)
            gb.wait()
            pltpu.sync_copy(rows_b, acc.at[ibuf_b.at[1]], add=True)
            return 0

        lax.fori_loop(0, cpw // 2, pair, 0)

        plsc.subcore_barrier()

        wdescs = []
        for t in range(per_sub):
            idx = lax.rem(sid + t * NS, nz)
            sl = pl.ds(idx * zr, zr)
            wdescs.append(
                pltpu.async_copy(acc.at[sl], out_hbm.at[cid].at[sl], sem_b))
        for dsc in wdescs:
            dsc.wait()

    return pl.kernel(
        body,
        out_type=jax.ShapeDtypeStruct((NC, n, d), jnp.float32),
        mesh=mesh,
        scratch_types=scratch,
    )


def _sc_scatter(m_pad, idx2):
    n = m_pad.shape[0] - MPAD
    d = m_pad.shape[1]
    return _make_sc_scatter(n, d, idx2.shape[0])(m_pad, idx2)


def kernel(feat, edge_index, W0, b0, Wih, Whh, bih, bhh):
    n, d = feat.shape
    e = edge_index.shape[1]
    src = edge_index[0]
    dst = edge_index[1]

    cpw = -(-e // (CH * NW))
    cpw = -(-cpw // 8) * 8
    e_pad = cpw * CH * NW
    pad_ar = jnp.arange(e_pad - e, dtype=jnp.int32)
    src = jnp.concatenate([src, n + pad_ar % MPAD])
    dst = jnp.concatenate([dst, pad_ar % n])
    idx2 = jnp.stack([src.reshape(-1, CH), dst.reshape(-1, CH)], axis=1)

    w0_t = W0.T
    wih_t = Wih.T
    whh_t = Whh.T
    b0_row = b0.reshape(1, -1)
    bih_row = bih.reshape(1, -1)
    bhh_row = bhh.reshape(1, -1)

    h = feat
    m_pad, gh = _init(h, w0_t, b0_row, whh_t, bhh_row)
    for k in range(STEPS):
        parts = _sc_scatter(m_pad, idx2)
        if k < STEPS - 1:
            h, m_pad = _boundary(parts, h, gh, wih_t, bih_row, w0_t, b0_row)
            gh = _gh(h, whh_t, bhh_row)
        else:
            h = _final_gru(parts, h, gh, wih_t, bih_row)
    return h


# speedup vs baseline: 1.0196x; 1.0196x over previous
import functools

import jax
import jax.numpy as jnp
from jax import lax
from jax.experimental import pallas as pl
from jax.experimental.pallas import tpu as pltpu
from jax.experimental.pallas import tpu_sc as plsc

STEPS = 3

NC = 2
NS = 16
NW = NC * NS
CH = 128
MPAD = 8


def _row_block(n):
    for r in (1024, 1000, 512, 500, 256, 250, 200, 128, 125, 80, 40, 8, 1):
        if n % r == 0:
            return r
    return 1


def _linear_pad_body(nb, h_ref, w_ref, b_ref, o_ref):
    i = pl.program_id(0)

    @pl.when(i < nb)
    def _():
        o_ref[...] = (
            jnp.dot(h_ref[...], w_ref[...], preferred_element_type=jnp.float32)
            + b_ref[...]
        )

    @pl.when(i >= nb)
    def _():
        o_ref[...] = jnp.zeros_like(o_ref)


def _linear_pad(h, w_t, b_row):
    n, d = h.shape
    dout = w_t.shape[1]
    r = _row_block(n)
    nb = n // r
    return pl.pallas_call(
        functools.partial(_linear_pad_body, nb),
        grid=(nb + 1,),
        in_specs=[
            pl.BlockSpec((r, d), lambda i: (jnp.minimum(i, nb - 1), 0)),
            pl.BlockSpec((d, dout), lambda i: (0, 0)),
            pl.BlockSpec((1, dout), lambda i: (0, 0)),
        ],
        out_specs=pl.BlockSpec((r, dout), lambda i: (i, 0)),
        out_shape=jax.ShapeDtypeStruct((n + MPAD, dout), jnp.float32),
    )(h, w_t, b_row)


def _gru_body(a_ref, h_ref, wih_ref, whh_ref, bih_ref, bhh_ref, o_ref):
    a = a_ref[0] + a_ref[1]
    h = h_ref[...]
    d = h.shape[1]
    gi = jnp.dot(a, wih_ref[...], preferred_element_type=jnp.float32) + bih_ref[...]
    gh = jnp.dot(h, whh_ref[...], preferred_element_type=jnp.float32) + bhh_ref[...]
    r = jax.nn.sigmoid(gi[:, :d] + gh[:, :d])
    z = jax.nn.sigmoid(gi[:, d:2 * d] + gh[:, d:2 * d])
    n = jnp.tanh(gi[:, 2 * d:] + r * gh[:, 2 * d:])
    o_ref[...] = (1.0 - z) * n + z * h


def _gru(a_parts, h, wih_t, whh_t, bih_row, bhh_row):
    n, d = h.shape
    r = _row_block(n)
    return pl.pallas_call(
        _gru_body,
        grid=(n // r,),
        in_specs=[
            pl.BlockSpec((2, r, d), lambda i: (0, i, 0)),
            pl.BlockSpec((r, d), lambda i: (i, 0)),
            pl.BlockSpec((d, 3 * d), lambda i: (0, 0)),
            pl.BlockSpec((d, 3 * d), lambda i: (0, 0)),
            pl.BlockSpec((1, 3 * d), lambda i: (0, 0)),
            pl.BlockSpec((1, 3 * d), lambda i: (0, 0)),
        ],
        out_specs=pl.BlockSpec((r, d), lambda i: (i, 0)),
        out_shape=jax.ShapeDtypeStruct((n, d), jnp.float32),
    )(a_parts, h, wih_t, whh_t, bih_row, bhh_row)


@functools.lru_cache(maxsize=None)
def _make_sc_scatter(n, d, nchunks):
    cpw = nchunks // NW
    zr = 8
    for c in (80, 64, 40, 32, 16, 8):
        if n % c == 0:
            zr = c
            break
    nz = n // zr
    per_sub = -(-nz // NS)

    mesh = plsc.VectorSubcoreMesh(core_axis_name="c", subcore_axis_name="s")
    scratch = [
        pltpu.VMEM((2, CH), jnp.int32),
        pltpu.VMEM((2, CH), jnp.int32),
        pltpu.VMEM((CH, d), jnp.float32),
        pltpu.VMEM((CH, d), jnp.float32),
        pltpu.VMEM_SHARED((n, d), jnp.float32),
        pltpu.VMEM((zr, d), jnp.float32),
        pltpu.SemaphoreType.DMA,
        pltpu.SemaphoreType.DMA,
    ]

    def body(m_hbm, idx_hbm, out_hbm, ibuf_a, ibuf_b, rows_a, rows_b,
             acc, zbuf, sem_a, sem_b):
        cid = lax.axis_index("c")
        sid = lax.axis_index("s")
        wid = cid * NS + sid

        zero16 = jnp.zeros((16,), jnp.float32)

        def zrow(i, _):
            for c in range(d // 16):
                zbuf[i, pl.ds(c * 16, 16)] = zero16
            return 0

        lax.fori_loop(0, zr, zrow, 0)

        zdescs = []
        for t in range(per_sub):
            idx = lax.rem(sid + t * NS, nz)
            zdescs.append(
                pltpu.async_copy(zbuf, acc.at[pl.ds(idx * zr, zr)], sem_a))
        for dsc in zdescs:
            dsc.wait()
        plsc.subcore_barrier()

        cbase = wid * cpw

        def pair(i, _):
            t = cbase + 2 * i
            ia = pltpu.async_copy(idx_hbm.at[t], ibuf_a, sem_a)
            ib = pltpu.async_copy(idx_hbm.at[t + 1], ibuf_b, sem_b)
            ia.wait()
            ga = pltpu.async_copy(m_hbm.at[ibuf_a.at[0]], rows_a, sem_a)
            ib.wait()
            gb = pltpu.async_copy(m_hbm.at[ibuf_b.at[0]], rows_b, sem_b)
            ga.wait()
            sa = pltpu.async_copy(rows_a, acc.at[ibuf_a.at[1]], sem_a,
                                  add=True)
            gb.wait()
            sb = pltpu.async_copy(rows_b, acc.at[ibuf_b.at[1]], sem_b,
                                  add=True)
            sa.wait()
            sb.wait()
            return 0

        lax.fori_loop(0, cpw // 2, pair, 0)

        plsc.subcore_barrier()

        wdescs = []
        for t in range(per_sub):
            idx = lax.rem(sid + t * NS, nz)
            sl = pl.ds(idx * zr, zr)
            wdescs.append(
                pltpu.async_copy(acc.at[sl], out_hbm.at[cid].at[sl], sem_b))
        for dsc in wdescs:
            dsc.wait()

    return pl.kernel(
        body,
        out_type=jax.ShapeDtypeStruct((NC, n, d), jnp.float32),
        mesh=mesh,
        scratch_types=scratch,
    )


def _sc_scatter(m_pad, idx2):
    n = m_pad.shape[0] - MPAD
    d = m_pad.shape[1]
    return _make_sc_scatter(n, d, idx2.shape[0])(m_pad, idx2)


def kernel(feat, edge_index, W0, b0, Wih, Whh, bih, bhh):
    n, d = feat.shape
    e = edge_index.shape[1]
    src = edge_index[0]
    dst = edge_index[1]

    cpw = -(-e // (CH * NW))
    cpw = -(-cpw // 8) * 8
    e_pad = cpw * CH * NW
    pad_ar = jnp.arange(e_pad - e, dtype=jnp.int32)
    src = jnp.concatenate([src, n + pad_ar % MPAD])
    dst = jnp.concatenate([dst, pad_ar % n])
    idx2 = jnp.stack([src.reshape(-1, CH), dst.reshape(-1, CH)], axis=1)

    w0_t = W0.T
    wih_t = Wih.T
    whh_t = Whh.T
    b0_row = b0.reshape(1, -1)
    bih_row = bih.reshape(1, -1)
    bhh_row = bhh.reshape(1, -1)

    h = feat
    for _ in range(STEPS):
        m_pad = _linear_pad(h, w0_t, b0_row)
        parts = _sc_scatter(m_pad, idx2)
        h = _gru(parts, h, wih_t, whh_t, bih_row, bhh_row)
    return h


# speedup vs baseline: 1.0544x; 1.0342x over previous
import functools

import jax
import jax.numpy as jnp
from jax import lax
from jax.experimental import pallas as pl
from jax.experimental.pallas import tpu as pltpu
from jax.experimental.pallas import tpu_sc as plsc

STEPS = 3

NC = 2
NS = 16
NW = NC * NS
CH = 128
MPAD = 8


def _row_block(n):
    for r in (1024, 1000, 512, 500, 256, 250, 200, 128, 125, 80, 40, 8, 1):
        if n % r == 0:
            return r
    return 1


def _linear_pad_body(nb, h_ref, w_ref, b_ref, o_ref):
    i = pl.program_id(0)

    @pl.when(i < nb)
    def _():
        o_ref[...] = (
            jnp.dot(h_ref[...], w_ref[...], preferred_element_type=jnp.float32)
            + b_ref[...]
        )

    @pl.when(i >= nb)
    def _():
        o_ref[...] = jnp.zeros_like(o_ref)


def _linear_pad(h, w_t, b_row):
    n, d = h.shape
    dout = w_t.shape[1]
    r = _row_block(n)
    nb = n // r
    return pl.pallas_call(
        functools.partial(_linear_pad_body, nb),
        grid=(nb + 1,),
        in_specs=[
            pl.BlockSpec((r, d), lambda i: (jnp.minimum(i, nb - 1), 0)),
            pl.BlockSpec((d, dout), lambda i: (0, 0)),
            pl.BlockSpec((1, dout), lambda i: (0, 0)),
        ],
        out_specs=pl.BlockSpec((r, dout), lambda i: (i, 0)),
        out_shape=jax.ShapeDtypeStruct((n + MPAD, dout), jnp.float32),
    )(h, w_t, b_row)


def _gru_math(a_ref, h, wih_ref, whh_ref, bih_ref, bhh_ref):
    a = a_ref[0] + a_ref[1]
    d = h.shape[1]
    gi = jnp.dot(a, wih_ref[...], preferred_element_type=jnp.float32) + bih_ref[...]
    gh = jnp.dot(h, whh_ref[...], preferred_element_type=jnp.float32) + bhh_ref[...]
    r = jax.nn.sigmoid(gi[:, :d] + gh[:, :d])
    z = jax.nn.sigmoid(gi[:, d:2 * d] + gh[:, d:2 * d])
    n = jnp.tanh(gi[:, 2 * d:] + r * gh[:, 2 * d:])
    return (1.0 - z) * n + z * h


def _gru_body(a_ref, h_ref, wih_ref, whh_ref, bih_ref, bhh_ref, o_ref):
    o_ref[...] = _gru_math(a_ref, h_ref[...], wih_ref, whh_ref, bih_ref,
                           bhh_ref)


def _boundary_body(nb, a_ref, h_ref, wih_ref, whh_ref, bih_ref, bhh_ref,
                   w0_ref, b0_ref, hn_ref, mn_ref):
    i = pl.program_id(0)
    hn = _gru_math(a_ref, h_ref[...], wih_ref, whh_ref, bih_ref, bhh_ref)
    hn_ref[...] = hn

    @pl.when(i < nb)
    def _():
        mn_ref[...] = (
            jnp.dot(hn, w0_ref[...], preferred_element_type=jnp.float32)
            + b0_ref[...]
        )

    @pl.when(i >= nb)
    def _():
        mn_ref[...] = jnp.zeros_like(mn_ref)


def _boundary(a_parts, h, wih_t, whh_t, bih_row, bhh_row, w0_t, b0_row):
    n, d = h.shape
    r = _row_block(n)
    nb = n // r
    clamp = lambda i: (jnp.minimum(i, nb - 1), 0)
    return pl.pallas_call(
        functools.partial(_boundary_body, nb),
        grid=(nb + 1,),
        in_specs=[
            pl.BlockSpec((2, r, d), lambda i: (0, jnp.minimum(i, nb - 1), 0)),
            pl.BlockSpec((r, d), clamp),
            pl.BlockSpec((d, 3 * d), lambda i: (0, 0)),
            pl.BlockSpec((d, 3 * d), lambda i: (0, 0)),
            pl.BlockSpec((1, 3 * d), lambda i: (0, 0)),
            pl.BlockSpec((1, 3 * d), lambda i: (0, 0)),
            pl.BlockSpec((d, d), lambda i: (0, 0)),
            pl.BlockSpec((1, d), lambda i: (0, 0)),
        ],
        out_specs=[
            pl.BlockSpec((r, d), clamp),
            pl.BlockSpec((r, d), lambda i: (i, 0)),
        ],
        out_shape=[
            jax.ShapeDtypeStruct((n, d), jnp.float32),
            jax.ShapeDtypeStruct((n + MPAD, d), jnp.float32),
        ],
    )(a_parts, h, wih_t, whh_t, bih_row, bhh_row, w0_t, b0_row)


def _gru(a_parts, h, wih_t, whh_t, bih_row, bhh_row):
    n, d = h.shape
    r = _row_block(n)
    return pl.pallas_call(
        _gru_body,
        grid=(n // r,),
        in_specs=[
            pl.BlockSpec((2, r, d), lambda i: (0, i, 0)),
            pl.BlockSpec((r, d), lambda i: (i, 0)),
            pl.BlockSpec((d, 3 * d), lambda i: (0, 0)),
            pl.BlockSpec((d, 3 * d), lambda i: (0, 0)),
            pl.BlockSpec((1, 3 * d), lambda i: (0, 0)),
            pl.BlockSpec((1, 3 * d), lambda i: (0, 0)),
        ],
        out_specs=pl.BlockSpec((r, d), lambda i: (i, 0)),
        out_shape=jax.ShapeDtypeStruct((n, d), jnp.float32),
    )(a_parts, h, wih_t, whh_t, bih_row, bhh_row)


@functools.lru_cache(maxsize=None)
def _make_sc_scatter(n, d, nchunks):
    cpw = nchunks // NW
    zr = 8
    for c in (80, 64, 40, 32, 16, 8):
        if n % c == 0:
            zr = c
            break
    nz = n // zr
    per_sub = -(-nz // NS)

    mesh = plsc.VectorSubcoreMesh(core_axis_name="c", subcore_axis_name="s")
    scratch = [
        pltpu.VMEM((2, CH), jnp.int32),
        pltpu.VMEM((2, CH), jnp.int32),
        pltpu.VMEM((CH, d), jnp.float32),
        pltpu.VMEM((CH, d), jnp.float32),
        pltpu.VMEM_SHARED((n, d), jnp.float32),
        pltpu.VMEM((zr, d), jnp.float32),
        pltpu.SemaphoreType.DMA,
        pltpu.SemaphoreType.DMA,
    ]

    def body(m_hbm, idx_hbm, out_hbm, ibuf_a, ibuf_b, rows_a, rows_b,
             acc, zbuf, sem_a, sem_b):
        cid = lax.axis_index("c")
        sid = lax.axis_index("s")
        wid = cid * NS + sid

        zero16 = jnp.zeros((16,), jnp.float32)

        def zrow(i, _):
            for c in range(d // 16):
                zbuf[i, pl.ds(c * 16, 16)] = zero16
            return 0

        lax.fori_loop(0, zr, zrow, 0)

        zdescs = []
        for t in range(per_sub):
            idx = lax.rem(sid + t * NS, nz)
            zdescs.append(
                pltpu.async_copy(zbuf, acc.at[pl.ds(idx * zr, zr)], sem_a))
        for dsc in zdescs:
            dsc.wait()
        plsc.subcore_barrier()

        cbase = wid * cpw

        def pair(i, _):
            t = cbase + 2 * i
            ia = pltpu.async_copy(idx_hbm.at[t], ibuf_a, sem_a)
            ib = pltpu.async_copy(idx_hbm.at[t + 1], ibuf_b, sem_b)
            ia.wait()
            ga = pltpu.async_copy(m_hbm.at[ibuf_a.at[0]], rows_a, sem_a)
            ib.wait()
            gb = pltpu.async_copy(m_hbm.at[ibuf_b.at[0]], rows_b, sem_b)
            ga.wait()
            sa = pltpu.async_copy(rows_a, acc.at[ibuf_a.at[1]], sem_a,
                                  add=True)
            gb.wait()
            sb = pltpu.async_copy(rows_b, acc.at[ibuf_b.at[1]], sem_b,
                                  add=True)
            sa.wait()
            sb.wait()
            return 0

        lax.fori_loop(0, cpw // 2, pair, 0)

        plsc.subcore_barrier()

        wdescs = []
        for t in range(per_sub):
            idx = lax.rem(sid + t * NS, nz)
            sl = pl.ds(idx * zr, zr)
            wdescs.append(
                pltpu.async_copy(acc.at[sl], out_hbm.at[cid].at[sl], sem_b))
        for dsc in wdescs:
            dsc.wait()

    return pl.kernel(
        body,
        out_type=jax.ShapeDtypeStruct((NC, n, d), jnp.float32),
        mesh=mesh,
        scratch_types=scratch,
    )


def _sc_scatter(m_pad, idx2):
    n = m_pad.shape[0] - MPAD
    d = m_pad.shape[1]
    return _make_sc_scatter(n, d, idx2.shape[0])(m_pad, idx2)


def kernel(feat, edge_index, W0, b0, Wih, Whh, bih, bhh):
    n, d = feat.shape
    e = edge_index.shape[1]
    src = edge_index[0]
    dst = edge_index[1]

    cpw = -(-e // (CH * NW))
    cpw = -(-cpw // 8) * 8
    e_pad = cpw * CH * NW
    pad_ar = jnp.arange(e_pad - e, dtype=jnp.int32)
    src = jnp.concatenate([src, n + pad_ar % MPAD])
    dst = jnp.concatenate([dst, pad_ar % n])
    idx2 = jnp.stack([src.reshape(-1, CH), dst.reshape(-1, CH)], axis=1)

    w0_t = W0.T
    wih_t = Wih.T
    whh_t = Whh.T
    b0_row = b0.reshape(1, -1)
    bih_row = bih.reshape(1, -1)
    bhh_row = bhh.reshape(1, -1)

    h = feat
    m_pad = _linear_pad(h, w0_t, b0_row)
    for k in range(STEPS):
        parts = _sc_scatter(m_pad, idx2)
        if k < STEPS - 1:
            h, m_pad = _boundary(parts, h, wih_t, whh_t, bih_row, bhh_row,
                                 w0_t, b0_row)
        else:
            h = _gru(parts, h, wih_t, whh_t, bih_row, bhh_row)
    return h
